# baseline (speedup 1.0000x reference)
import jax
import jax.numpy as jnp
from jax.experimental import pallas as pl

_B, _H, _W, _C = 4096, 22, 22, 3
_HW = _H * _W
_NCLS = 11
_BBLK = 128


def _onehot_body(x_ref, o_ref):
    idx = x_ref[...]
    cls = jax.lax.broadcasted_iota(jnp.int32, (1, _NCLS, 1), 1)
    o_ref[...] = (idx[:, None, :] == cls).astype(jnp.float32)


def kernel(x):
    x0 = x[..., 0].reshape(_B, _HW).astype(jnp.int32)
    out = pl.pallas_call(
        _onehot_body,
        grid=(_B // _BBLK,),
        in_specs=[pl.BlockSpec((_BBLK, _HW), lambda i: (i, 0))],
        out_specs=pl.BlockSpec((_BBLK, _NCLS, _HW), lambda i: (i, 0, 0)),
        out_shape=jax.ShapeDtypeStruct((_B, _NCLS, _HW), jnp.float32),
    )(x0)
    return out.reshape(_B, _NCLS, _H, _W)


# speedup vs baseline: 7.9886x; 7.9886x over previous
import jax
import jax.numpy as jnp
from jax.experimental import pallas as pl

_B, _H, _W, _C = 4096, 22, 22, 3
_NCLS = 11


def _onehot_body(x_ref, o_ref):
    c = pl.program_id(0)
    xv = x_ref[:, 0]
    o_ref[0] = (xv == c).astype(jnp.float32)


def kernel(x):
    xt = jnp.transpose(x, (1, 3, 2, 0))
    ot = pl.pallas_call(
        _onehot_body,
        grid=(_NCLS,),
        in_specs=[pl.BlockSpec((_H, 1, _W, _B), lambda i: (0, 0, 0, 0))],
        out_specs=pl.BlockSpec((1, _H, _W, _B), lambda i: (i, 0, 0, 0)),
        out_shape=jax.ShapeDtypeStruct((_NCLS, _H, _W, _B), jnp.float32),
    )(xt)
    return jnp.transpose(ot, (3, 0, 1, 2))
